# baseline (speedup 1.0000x reference)
import functools

import jax
import jax.numpy as jnp
from jax import lax
from jax.experimental import pallas as pl
from jax.experimental.pallas import tpu as pltpu
from jax.experimental.pallas import tpu_sc as plsc

NC = 2
NS = 16
NW = NC * NS
CH = 128


def _sc_gather(idx3, table, n_ch, per_w, hidden):
    mesh = plsc.VectorSubcoreMesh(
        core_axis_name="c", subcore_axis_name="s",
        num_cores=NC, num_subcores=NS)

    @functools.partial(
        pl.kernel,
        out_type=jax.ShapeDtypeStruct((NW * per_w, hidden), jnp.float32),
        mesh=mesh,
        scratch_types=[
            pltpu.VMEM((n_ch, CH), jnp.int32),
            pltpu.VMEM((per_w, hidden), jnp.float32),
            pltpu.SemaphoreType.DMA,
        ],
        compiler_params=pltpu.CompilerParams(use_tc_tiling_on_sc=False),
    )
    def body(idx_hbm, table_hbm, out_hbm, idx_v, rows_v, sem):
        wid = lax.axis_index("s") * NC + lax.axis_index("c")
        pltpu.sync_copy(idx_hbm.at[wid], idx_v)
        copies = [
            pltpu.async_copy(
                table_hbm.at[idx_v.at[j]],
                rows_v.at[pl.ds(j * CH, CH)],
                sem,
            )
            for j in range(n_ch)
        ]
        for c in copies:
            c.wait()
        pltpu.sync_copy(rows_v, out_hbm.at[pl.ds(wid * per_w, per_w)])

    return body(idx3, table)


def _mlp_body(g_a_ref, g_b_ref, wa_ref, wb_ref, b1_ref, w2_ref, b2_ref, o_ref):
    a = g_a_ref[0]
    b = g_b_ref[0]
    h = jnp.dot(a, wa_ref[...], preferred_element_type=jnp.float32)
    h = h + jnp.dot(b, wb_ref[...], preferred_element_type=jnp.float32)
    h = jnp.maximum(h + b1_ref[...], 0.0)
    o_ref[...] = (
        jnp.dot(h, w2_ref[...], preferred_element_type=jnp.float32)
        + b2_ref[...]
    )


def kernel(x, emb, W1, b1, W2, b2):
    B = x.shape[0]
    H = emb.shape[1]
    L = W2.shape[0]

    total = 2 * B
    per_w = total // NW
    n_ch = per_w // CH

    idx3 = x.T.reshape(NW, n_ch, CH)
    g = _sc_gather(idx3, emb, n_ch, per_w, H)
    g3 = g.reshape(2, B, H)

    Wa = W1[:, :H].T
    Wb = W1[:, H:].T
    W2T = W2.T

    BB = 2048
    grid = (B // BB,)
    out = pl.pallas_call(
        _mlp_body,
        grid=grid,
        in_specs=[
            pl.BlockSpec((1, BB, H), lambda i: (0, i, 0)),
            pl.BlockSpec((1, BB, H), lambda i: (1, i, 0)),
            pl.BlockSpec((H, H), lambda i: (0, 0)),
            pl.BlockSpec((H, H), lambda i: (0, 0)),
            pl.BlockSpec((1, H), lambda i: (0, 0)),
            pl.BlockSpec((H, L), lambda i: (0, 0)),
            pl.BlockSpec((1, L), lambda i: (0, 0)),
        ],
        out_specs=pl.BlockSpec((BB, L), lambda i: (i, 0)),
        out_shape=jax.ShapeDtypeStruct((B, L), jnp.float32),
    )(g3, g3, Wa, Wb, b1.reshape(1, H), W2T, b2.reshape(1, L))
    return out
